# Initial kernel scaffold; baseline (speedup 1.0000x reference)
import jax
import jax.numpy as jnp
from jax.experimental import pallas as pl


def kernel(x, day_embed):
    raise NotImplementedError("write your pallas kernel here")


# speedup vs baseline: 3.0875x; 3.0875x over previous
import functools

import jax
import jax.numpy as jnp
from jax import lax
from jax.experimental import pallas as pl
from jax.experimental.pallas import tpu as pltpu
from jax.experimental.pallas import tpu_sc as plsc

DAY = 288
D = 128
B_TOTAL = 4096 * 200
NW = 32
B_PER_W = B_TOTAL // NW
G = 128
NGRP = B_PER_W // G
NBUF = 4
L = 16


def _make_sc_call():
    mesh = plsc.VectorSubcoreMesh(core_axis_name="c", subcore_axis_name="s")

    @functools.partial(
        pl.kernel,
        out_type=jax.ShapeDtypeStruct((B_TOTAL, D), jnp.float32),
        mesh=mesh,
        scratch_types=(
            [pltpu.VMEM((B_PER_W,), jnp.float32)]
            + [pltpu.VMEM((NBUF, G), jnp.int32)]
            + [pltpu.VMEM((G, D), jnp.float32) for _ in range(NBUF)]
            + [pltpu.SemaphoreType.DMA for _ in range(2 * NBUF)]
        ),
    )
    def sc_embed(x_hbm, table_hbm, out_hbm, x_v, idx_v, *rest):
        rows = rest[:NBUF]
        gsem = rest[NBUF:2 * NBUF]
        wsem = rest[2 * NBUF:]

        wid = lax.axis_index("s") * 2 + lax.axis_index("c")
        base = wid * B_PER_W

        pltpu.sync_copy(x_hbm.at[pl.ds(base, B_PER_W)], x_v)

        def compute_idx(g, b):
            for i in range(G // L):
                xv = x_v[pl.ds(g * G + i * L, L)]
                idx_v[b, pl.ds(i * L, L)] = (xv * float(DAY)).astype(jnp.int32)

        def gather(b):
            return pltpu.make_async_copy(table_hbm.at[idx_v.at[b]], rows[b], gsem[b])

        def write(b, g):
            return pltpu.make_async_copy(
                rows[b], out_hbm.at[pl.ds(base + g * G, G)], wsem[b])

        for b in range(NBUF):
            compute_idx(b, b)
            gather(b).start()

        def body(go, _):
            for b in range(NBUF):
                gg = go * NBUF + b
                gather(b).wait()
                w = write(b, gg)
                w.start()
                compute_idx(gg + NBUF, b)
                w.wait()
                gather(b).start()
            return _

        lax.fori_loop(0, (NGRP - NBUF) // NBUF, body, None)

        for b in range(NBUF):
            gather(b).wait()
            write(b, NGRP - NBUF + b).start()
        for b in range(NBUF):
            write(b, NGRP - NBUF + b).wait()

    return sc_embed


_sc_embed = _make_sc_call()


@jax.jit
def kernel(x, day_embed):
    out = _sc_embed(x.reshape(B_TOTAL), day_embed)
    return out.reshape(x.shape[0], x.shape[1], D)


# speedup vs baseline: 15.9492x; 5.1657x over previous
import functools

import jax
import jax.numpy as jnp
from jax import lax
from jax.experimental import pallas as pl
from jax.experimental.pallas import tpu as pltpu
from jax.experimental.pallas import tpu_sc as plsc

DAY = 288
D = 128
B_TOTAL = 4096 * 200
NW = 32
B_PER_W = B_TOTAL // NW
G = 128
NGRP = B_PER_W // G
NBUF = 4
L = 16


def _make_sc_call():
    mesh = plsc.VectorSubcoreMesh(core_axis_name="c", subcore_axis_name="s")

    @functools.partial(
        pl.kernel,
        out_type=jax.ShapeDtypeStruct((B_TOTAL, D), jnp.float32),
        mesh=mesh,
        scratch_types=(
            [pltpu.VMEM_SHARED((DAY, D), jnp.float32)]
            + [pltpu.VMEM((B_PER_W,), jnp.float32)]
            + [pltpu.VMEM((NBUF, G), jnp.int32)]
            + [pltpu.VMEM((G, D), jnp.float32) for _ in range(NBUF)]
            + [pltpu.SemaphoreType.DMA for _ in range(2 * NBUF)]
        ),
    )
    def sc_embed(x_hbm, table_hbm, out_hbm, table_sp, x_v, idx_v, *rest):
        rows = rest[:NBUF]
        gsem = rest[NBUF:2 * NBUF]
        wsem = rest[2 * NBUF:]

        wid = lax.axis_index("s") * 2 + lax.axis_index("c")
        base = wid * B_PER_W

        @pl.when(lax.axis_index("s") == 0)
        def _():
            pltpu.sync_copy(table_hbm, table_sp)

        pltpu.sync_copy(x_hbm.at[pl.ds(base, B_PER_W)], x_v)
        plsc.subcore_barrier()

        def compute_idx(g, b):
            for i in range(G // L):
                xv = x_v[pl.ds(g * G + i * L, L)]
                idx_v[b, pl.ds(i * L, L)] = (xv * float(DAY)).astype(jnp.int32)

        def gather(b):
            return pltpu.make_async_copy(table_sp.at[idx_v.at[b]], rows[b], gsem[b])

        def write(b, g):
            return pltpu.make_async_copy(
                rows[b], out_hbm.at[pl.ds(base + g * G, G)], wsem[b])

        for b in range(NBUF):
            compute_idx(b, b)
            gather(b).start()

        def body(go, _):
            for b in range(NBUF):
                gg = go * NBUF + b
                gather(b).wait()
                w = write(b, gg)
                w.start()
                compute_idx(gg + NBUF, b)
                w.wait()
                gather(b).start()
            return _

        lax.fori_loop(0, (NGRP - NBUF) // NBUF, body, None)

        for b in range(NBUF):
            gather(b).wait()
            write(b, NGRP - NBUF + b).start()
        for b in range(NBUF):
            write(b, NGRP - NBUF + b).wait()

    return sc_embed


_sc_embed = _make_sc_call()


@jax.jit
def kernel(x, day_embed):
    out = _sc_embed(x.reshape(B_TOTAL), day_embed)
    return out.reshape(x.shape[0], x.shape[1], D)
